# baseline (speedup 1.0000x reference)
import functools
import math
from fractions import Fraction

import numpy as np
import jax
import jax.numpy as jnp
from jax import lax
from jax.experimental import pallas as pl
from jax.experimental.pallas import tpu as pltpu
from jax.experimental.pallas import tpu_sc as plsc

_LS1 = [0, 1, 2]
_LS2 = [0, 1, 2]
_BATCH = 65536


def _cg_su2(j1, m1, j2, m2, j3, m3):
    if m3 != m1 + m2:
        return 0.0
    f = math.factorial
    vmin = int(max(-j1 + j2 + m3, -j1 + m1, 0))
    vmax = int(min(j2 + j3 + m1, j3 - j1 + j2, j3 + m3))
    C = math.sqrt((2 * j3 + 1) * Fraction(
        f(j3 + j1 - j2) * f(j3 - j1 + j2) * f(j1 + j2 - j3) * f(j3 + m3) * f(j3 - m3),
        f(j1 + j2 + j3 + 1) * f(j1 - m1) * f(j1 + m1) * f(j2 - m2) * f(j2 + m2)))
    S = 0
    for v in range(vmin, vmax + 1):
        S += (-1) ** (v + j2 + m2) * Fraction(
            f(j2 + j3 + m1 - v) * f(j1 - m1 + v),
            f(v) * f(j3 - j1 + j2 - v) * f(j3 + m3 - v) * f(v + j1 - j2 - m3))
    return C * float(S)


def _cg_change_basis(l):
    q = np.zeros((2 * l + 1, 2 * l + 1), dtype=np.complex128)
    for m in range(-l, 0):
        q[l + m, l + abs(m)] = 1.0 / math.sqrt(2)
        q[l + m, l - abs(m)] = -1j / math.sqrt(2)
    q[l, l] = 1.0
    for m in range(1, l + 1):
        q[l + m, l + abs(m)] = (-1) ** m / math.sqrt(2)
        q[l + m, l - abs(m)] = 1j * (-1) ** m / math.sqrt(2)
    return (-1j) ** l * q


def _cg_real_w3j(l1, l2, l3):
    C = np.zeros((2 * l1 + 1, 2 * l2 + 1, 2 * l3 + 1), dtype=np.complex128)
    for m1 in range(-l1, l1 + 1):
        for m2 in range(-l2, l2 + 1):
            m3 = m1 + m2
            if abs(m3) <= l3:
                C[l1 + m1, l2 + m2, l3 + m3] = _cg_su2(l1, m1, l2, m2, l3, m3)
    Q1, Q2, Q3 = _cg_change_basis(l1), _cg_change_basis(l2), _cg_change_basis(l3)
    C = np.einsum('ij,kl,mn,ikn->jlm', Q1, Q2, np.conj(Q3.T), C)
    C = np.real(C)
    return C / np.linalg.norm(C)


def _cg_blocks(ls1, ls2):
    lmax2 = max(ls2)
    cb_layout = {}
    off1 = 0
    for l1 in ls1:
        off2 = 0
        for l2 in ls2:
            for l3 in range(abs(l1 - l2), l1 + l2 + 1):
                cb_layout.setdefault(l3, []).append((l1, l2, off1, off2))
            off2 += 2 * l2 + 1
        off1 += 2 * l1 + 1
    blocks = []
    row_offset = 0
    for l3 in sorted(cb_layout.keys()):
        mults = sorted(cb_layout[l3], key=lambda x: x[0] * lmax2 + x[1])
        for (l1, l2, o1, o2) in mults:
            cb = _cg_real_w3j(l1, l2, l3)
            rows = {}
            for m3 in range(2 * l3 + 1):
                terms = []
                for m2 in range(2 * l2 + 1):
                    for m1 in range(2 * l1 + 1):
                        c = cb[m1, m2, m3]
                        if abs(c) < 1e-12:
                            continue
                        terms.append((m1 + o1, m2 + o2,
                                      float(c) * math.sqrt(2 * l3 + 1)))
                if terms:
                    rows[m3 + row_offset] = terms
            blocks.append((l1, l2, rows))
            row_offset += 2 * l3 + 1
    return row_offset, blocks


_HEIGHT, _BLOCKS = _cg_blocks(_LS1, _LS2)
_DIM1 = sum(2 * l + 1 for l in _LS1)
_DIM2 = sum(2 * l + 1 for l in _LS2)

_NC, _NS, _L = 2, 16, 16
_NW = _NC * _NS
_RBLK = 512
_ROWS_PER_W = _BATCH // _NW
_NBLK = _ROWS_PER_W // _RBLK


def _tp_body(in1_hbm, in2_hbm, out_hbm, a_v, b_v, oc_v):
    wid = lax.axis_index("s") * _NC + lax.axis_index("c")
    base = wid * _ROWS_PER_W

    def block(bi, carry):
        rbase = base + bi * _RBLK
        pltpu.sync_copy(in1_hbm.at[pl.ds(rbase * _DIM1, _RBLK * _DIM1)], a_v)
        pltpu.sync_copy(in2_hbm.at[pl.ds(rbase * _DIM2, _RBLK * _DIM2)], b_v)

        @plsc.parallel_loop(0, _RBLK // _L, 1)
        def group(g):
            rows = g * _L + lax.iota(jnp.int32, _L)
            rows1 = rows * _DIM1
            rows2 = rows * _DIM2
            rowso = rows * _HEIGHT
            a = [plsc.load_gather(a_v, [rows1 + i]) for i in range(_DIM1)]
            b = [plsc.load_gather(b_v, [rows2 + j]) for j in range(_DIM2)]
            for (_l1, _l2, orows) in _BLOCKS:
                for o, terms in orows.items():
                    acc = None
                    for (i1, i2, val) in terms:
                        t = (a[i1] * b[i2]) * jnp.float32(val)
                        acc = t if acc is None else acc + t
                    plsc.store_scatter(oc_v, [rowso + o], acc)

        pltpu.sync_copy(oc_v, out_hbm.at[pl.ds(rbase * _HEIGHT, _RBLK * _HEIGHT)])
        return carry

    lax.fori_loop(0, _NBLK, block, 0)


@jax.jit
def kernel(in1, in2):
    mesh = plsc.VectorSubcoreMesh(core_axis_name="c", subcore_axis_name="s")
    f = functools.partial(
        pl.kernel,
        mesh=mesh,
        compiler_params=pltpu.CompilerParams(
            needs_layout_passes=False,
            disable_bounds_checks=True,
            skip_device_barrier=True,
        ),
        out_type=jax.ShapeDtypeStruct((_BATCH * _HEIGHT,), jnp.float32),
        scratch_types=[
            pltpu.VMEM((_RBLK * _DIM1,), jnp.float32),
            pltpu.VMEM((_RBLK * _DIM2,), jnp.float32),
            pltpu.VMEM((_RBLK * _HEIGHT,), jnp.float32),
        ],
    )(_tp_body)
    return f(in1.reshape(-1), in2.reshape(-1)).reshape(_BATCH, _HEIGHT)


# speedup vs baseline: 1.2257x; 1.2257x over previous
import functools
import math
from fractions import Fraction

import numpy as np
import jax
import jax.numpy as jnp
from jax import lax
from jax.experimental import pallas as pl
from jax.experimental.pallas import tpu as pltpu
from jax.experimental.pallas import tpu_sc as plsc

_LS1 = [0, 1, 2]
_LS2 = [0, 1, 2]
_BATCH = 65536


def _cg_su2(j1, m1, j2, m2, j3, m3):
    if m3 != m1 + m2:
        return 0.0
    f = math.factorial
    vmin = int(max(-j1 + j2 + m3, -j1 + m1, 0))
    vmax = int(min(j2 + j3 + m1, j3 - j1 + j2, j3 + m3))
    C = math.sqrt((2 * j3 + 1) * Fraction(
        f(j3 + j1 - j2) * f(j3 - j1 + j2) * f(j1 + j2 - j3) * f(j3 + m3) * f(j3 - m3),
        f(j1 + j2 + j3 + 1) * f(j1 - m1) * f(j1 + m1) * f(j2 - m2) * f(j2 + m2)))
    S = 0
    for v in range(vmin, vmax + 1):
        S += (-1) ** (v + j2 + m2) * Fraction(
            f(j2 + j3 + m1 - v) * f(j1 - m1 + v),
            f(v) * f(j3 - j1 + j2 - v) * f(j3 + m3 - v) * f(v + j1 - j2 - m3))
    return C * float(S)


def _cg_change_basis(l):
    q = np.zeros((2 * l + 1, 2 * l + 1), dtype=np.complex128)
    for m in range(-l, 0):
        q[l + m, l + abs(m)] = 1.0 / math.sqrt(2)
        q[l + m, l - abs(m)] = -1j / math.sqrt(2)
    q[l, l] = 1.0
    for m in range(1, l + 1):
        q[l + m, l + abs(m)] = (-1) ** m / math.sqrt(2)
        q[l + m, l - abs(m)] = 1j * (-1) ** m / math.sqrt(2)
    return (-1j) ** l * q


def _cg_real_w3j(l1, l2, l3):
    C = np.zeros((2 * l1 + 1, 2 * l2 + 1, 2 * l3 + 1), dtype=np.complex128)
    for m1 in range(-l1, l1 + 1):
        for m2 in range(-l2, l2 + 1):
            m3 = m1 + m2
            if abs(m3) <= l3:
                C[l1 + m1, l2 + m2, l3 + m3] = _cg_su2(l1, m1, l2, m2, l3, m3)
    Q1, Q2, Q3 = _cg_change_basis(l1), _cg_change_basis(l2), _cg_change_basis(l3)
    C = np.einsum('ij,kl,mn,ikn->jlm', Q1, Q2, np.conj(Q3.T), C)
    C = np.real(C)
    return C / np.linalg.norm(C)


def _cg_blocks(ls1, ls2):
    lmax2 = max(ls2)
    cb_layout = {}
    off1 = 0
    for l1 in ls1:
        off2 = 0
        for l2 in ls2:
            for l3 in range(abs(l1 - l2), l1 + l2 + 1):
                cb_layout.setdefault(l3, []).append((l1, l2, off1, off2))
            off2 += 2 * l2 + 1
        off1 += 2 * l1 + 1
    blocks = []
    row_offset = 0
    for l3 in sorted(cb_layout.keys()):
        mults = sorted(cb_layout[l3], key=lambda x: x[0] * lmax2 + x[1])
        for (l1, l2, o1, o2) in mults:
            cb = _cg_real_w3j(l1, l2, l3)
            rows = {}
            for m3 in range(2 * l3 + 1):
                terms = []
                for m2 in range(2 * l2 + 1):
                    for m1 in range(2 * l1 + 1):
                        c = cb[m1, m2, m3]
                        if abs(c) < 1e-12:
                            continue
                        terms.append((m1 + o1, m2 + o2,
                                      float(c) * math.sqrt(2 * l3 + 1)))
                if terms:
                    rows[m3 + row_offset] = terms
            blocks.append((l1, l2, rows))
            row_offset += 2 * l3 + 1
    return row_offset, blocks


_HEIGHT, _BLOCKS = _cg_blocks(_LS1, _LS2)
_DIM1 = sum(2 * l + 1 for l in _LS1)
_DIM2 = sum(2 * l + 1 for l in _LS2)

_NC, _NS, _L = 2, 16, 16
_NW = _NC * _NS
_RBLK = 512
_ROWS_PER_W = _BATCH // _NW
_NBLK = _ROWS_PER_W // _RBLK
_CHUNKS = [0, 16, 32, 48, 64, 65]


def _pair_blocks(blocks):
    merged = {}
    for (l1, l2, orows) in blocks:
        dst = merged.setdefault((l1, l2), {})
        for o, terms in orows.items():
            dst.setdefault(o, []).extend(terms)
    out = []
    for (l1, l2), orows in merged.items():
        pairs = []
        for terms in orows.values():
            for (i1, i2, _val) in terms:
                if (i1, i2) not in pairs:
                    pairs.append((i1, i2))
        out.append((pairs, orows))
    return out


_PAIR_BLOCKS = _pair_blocks(_BLOCKS)


def _tp_body(in1_hbm, in2_hbm, out_hbm, a_v, b_v, oc_v, op_v):
    wid = lax.axis_index("s") * _NC + lax.axis_index("c")
    base = wid * _ROWS_PER_W

    def block(bi, carry):
        rbase = base + bi * _RBLK
        pltpu.sync_copy(in1_hbm.at[pl.ds(rbase * _DIM1, _RBLK * _DIM1)], a_v)
        pltpu.sync_copy(in2_hbm.at[pl.ds(rbase * _DIM2, _RBLK * _DIM2)], b_v)

        @plsc.parallel_loop(0, _RBLK // _L, 1)
        def group(g):
            rows = g * _L + lax.iota(jnp.int32, _L)
            rows1 = rows * _DIM1
            rows2 = rows * _DIM2
            rowso = rows * _HEIGHT
            a = [plsc.load_gather(a_v, [rows1 + i]) for i in range(_DIM1)]
            b = [plsc.load_gather(b_v, [rows2 + j]) for j in range(_DIM2)]
            for (pairs, orows) in _PAIR_BLOCKS:
                prod = {p: a[p[0]] * b[p[1]] for p in pairs}
                for o, terms in orows.items():
                    acc = None
                    for (i1, i2, val) in terms:
                        t = prod[(i1, i2)] * jnp.float32(val)
                        acc = t if acc is None else acc + t
                    plsc.store_scatter(oc_v, [rowso + o], acc)

        @plsc.parallel_loop(0, _RBLK, 1)
        def repack(r):
            for c in _CHUNKS:
                op_v[r, pl.ds(c, _L)] = oc_v[pl.ds(r * _HEIGHT + c, _L)]

        pltpu.sync_copy(op_v, out_hbm.at[pl.ds(rbase, _RBLK)])
        return carry

    lax.fori_loop(0, _NBLK, block, 0)


@jax.jit
def kernel(in1, in2):
    mesh = plsc.VectorSubcoreMesh(core_axis_name="c", subcore_axis_name="s")
    f = functools.partial(
        pl.kernel,
        mesh=mesh,
        compiler_params=pltpu.CompilerParams(
            needs_layout_passes=False,
            disable_bounds_checks=True,
            skip_device_barrier=True,
        ),
        out_type=jax.ShapeDtypeStruct((_BATCH, _HEIGHT), jnp.float32),
        scratch_types=[
            pltpu.VMEM((_RBLK * _DIM1,), jnp.float32),
            pltpu.VMEM((_RBLK * _DIM2,), jnp.float32),
            pltpu.VMEM((_RBLK * _HEIGHT,), jnp.float32),
            pltpu.VMEM((_RBLK, _HEIGHT), jnp.float32),
        ],
    )(_tp_body)
    return f(in1.reshape(-1), in2.reshape(-1))


# speedup vs baseline: 1.3183x; 1.0756x over previous
import functools
import math
from fractions import Fraction

import numpy as np
import jax
import jax.numpy as jnp
from jax import lax
from jax.experimental import pallas as pl
from jax.experimental.pallas import tpu as pltpu
from jax.experimental.pallas import tpu_sc as plsc

_LS1 = [0, 1, 2]
_LS2 = [0, 1, 2]
_BATCH = 65536


def _cg_su2(j1, m1, j2, m2, j3, m3):
    if m3 != m1 + m2:
        return 0.0
    f = math.factorial
    vmin = int(max(-j1 + j2 + m3, -j1 + m1, 0))
    vmax = int(min(j2 + j3 + m1, j3 - j1 + j2, j3 + m3))
    C = math.sqrt((2 * j3 + 1) * Fraction(
        f(j3 + j1 - j2) * f(j3 - j1 + j2) * f(j1 + j2 - j3) * f(j3 + m3) * f(j3 - m3),
        f(j1 + j2 + j3 + 1) * f(j1 - m1) * f(j1 + m1) * f(j2 - m2) * f(j2 + m2)))
    S = 0
    for v in range(vmin, vmax + 1):
        S += (-1) ** (v + j2 + m2) * Fraction(
            f(j2 + j3 + m1 - v) * f(j1 - m1 + v),
            f(v) * f(j3 - j1 + j2 - v) * f(j3 + m3 - v) * f(v + j1 - j2 - m3))
    return C * float(S)


def _cg_change_basis(l):
    q = np.zeros((2 * l + 1, 2 * l + 1), dtype=np.complex128)
    for m in range(-l, 0):
        q[l + m, l + abs(m)] = 1.0 / math.sqrt(2)
        q[l + m, l - abs(m)] = -1j / math.sqrt(2)
    q[l, l] = 1.0
    for m in range(1, l + 1):
        q[l + m, l + abs(m)] = (-1) ** m / math.sqrt(2)
        q[l + m, l - abs(m)] = 1j * (-1) ** m / math.sqrt(2)
    return (-1j) ** l * q


def _cg_real_w3j(l1, l2, l3):
    C = np.zeros((2 * l1 + 1, 2 * l2 + 1, 2 * l3 + 1), dtype=np.complex128)
    for m1 in range(-l1, l1 + 1):
        for m2 in range(-l2, l2 + 1):
            m3 = m1 + m2
            if abs(m3) <= l3:
                C[l1 + m1, l2 + m2, l3 + m3] = _cg_su2(l1, m1, l2, m2, l3, m3)
    Q1, Q2, Q3 = _cg_change_basis(l1), _cg_change_basis(l2), _cg_change_basis(l3)
    C = np.einsum('ij,kl,mn,ikn->jlm', Q1, Q2, np.conj(Q3.T), C)
    C = np.real(C)
    return C / np.linalg.norm(C)


def _cg_blocks(ls1, ls2):
    lmax2 = max(ls2)
    cb_layout = {}
    off1 = 0
    for l1 in ls1:
        off2 = 0
        for l2 in ls2:
            for l3 in range(abs(l1 - l2), l1 + l2 + 1):
                cb_layout.setdefault(l3, []).append((l1, l2, off1, off2))
            off2 += 2 * l2 + 1
        off1 += 2 * l1 + 1
    blocks = []
    row_offset = 0
    for l3 in sorted(cb_layout.keys()):
        mults = sorted(cb_layout[l3], key=lambda x: x[0] * lmax2 + x[1])
        for (l1, l2, o1, o2) in mults:
            cb = _cg_real_w3j(l1, l2, l3)
            rows = {}
            for m3 in range(2 * l3 + 1):
                terms = []
                for m2 in range(2 * l2 + 1):
                    for m1 in range(2 * l1 + 1):
                        c = cb[m1, m2, m3]
                        if abs(c) < 1e-12:
                            continue
                        terms.append((m1 + o1, m2 + o2,
                                      float(c) * math.sqrt(2 * l3 + 1)))
                if terms:
                    rows[m3 + row_offset] = terms
            blocks.append((l1, l2, rows))
            row_offset += 2 * l3 + 1
    return row_offset, blocks


_HEIGHT, _BLOCKS = _cg_blocks(_LS1, _LS2)
_DIM1 = sum(2 * l + 1 for l in _LS1)
_DIM2 = sum(2 * l + 1 for l in _LS2)

_NC, _NS, _L = 2, 16, 16
_NW = _NC * _NS
_RBLK = 256
_ROWS_PER_W = _BATCH // _NW
_NBLK = _ROWS_PER_W // _RBLK
_CHUNKS = [0, 16, 32, 48, 64, 65]


def _pair_blocks(blocks):
    merged = {}
    for (l1, l2, orows) in blocks:
        dst = merged.setdefault((l1, l2), {})
        for o, terms in orows.items():
            dst.setdefault(o, []).extend(terms)
    out = []
    for (l1, l2), orows in merged.items():
        pairs = []
        for terms in orows.values():
            for (i1, i2, _val) in terms:
                if (i1, i2) not in pairs:
                    pairs.append((i1, i2))
        out.append((pairs, orows))
    return out


_PAIR_BLOCKS = _pair_blocks(_BLOCKS)


def _tp_body(in1_hbm, in2_hbm, out_hbm, a_v, b_v, oc_v, op_v):
    wid = lax.axis_index("s") * _NC + lax.axis_index("c")
    base = wid * _ROWS_PER_W

    def block(bi, carry):
        rbase = base + bi * _RBLK
        pltpu.sync_copy(in1_hbm.at[pl.ds(rbase, _RBLK)], a_v)
        pltpu.sync_copy(in2_hbm.at[pl.ds(rbase, _RBLK)], b_v)

        @plsc.parallel_loop(0, _RBLK // _L, 1)
        def group(g):
            rows = g * _L + lax.iota(jnp.int32, _L)
            rowso = rows * _HEIGHT
            a = [plsc.load_gather(a_v, [rows, rows * 0 + i]) for i in range(_DIM1)]
            b = [plsc.load_gather(b_v, [rows, rows * 0 + j]) for j in range(_DIM2)]
            for (pairs, orows) in _PAIR_BLOCKS:
                prod = {p: a[p[0]] * b[p[1]] for p in pairs}
                for o, terms in orows.items():
                    acc = None
                    for (i1, i2, val) in terms:
                        t = prod[(i1, i2)] * jnp.float32(val)
                        acc = t if acc is None else acc + t
                    plsc.store_scatter(oc_v, [rowso + o], acc)

        @plsc.parallel_loop(0, _RBLK, 1)
        def repack(r):
            for c in _CHUNKS:
                op_v[r, pl.ds(c, _L)] = oc_v[pl.ds(r * _HEIGHT + c, _L)]

        pltpu.sync_copy(op_v, out_hbm.at[pl.ds(rbase, _RBLK)])
        return carry

    lax.fori_loop(0, _NBLK, block, 0)


@jax.jit
def kernel(in1, in2):
    mesh = plsc.VectorSubcoreMesh(core_axis_name="c", subcore_axis_name="s")
    f = functools.partial(
        pl.kernel,
        mesh=mesh,
        compiler_params=pltpu.CompilerParams(
            needs_layout_passes=False,
            disable_bounds_checks=True,
            skip_device_barrier=True,
        ),
        out_type=jax.ShapeDtypeStruct((_BATCH, _HEIGHT), jnp.float32),
        scratch_types=[
            pltpu.VMEM((_RBLK, _DIM1), jnp.float32),
            pltpu.VMEM((_RBLK, _DIM2), jnp.float32),
            pltpu.VMEM((_RBLK * _HEIGHT,), jnp.float32),
            pltpu.VMEM((_RBLK, _HEIGHT), jnp.float32),
        ],
    )(_tp_body)
    return f(in1, in2)


# speedup vs baseline: 1.3339x; 1.0118x over previous
import functools
import math
from fractions import Fraction

import numpy as np
import jax
import jax.numpy as jnp
from jax import lax
from jax.experimental import pallas as pl
from jax.experimental.pallas import tpu as pltpu
from jax.experimental.pallas import tpu_sc as plsc

_LS1 = [0, 1, 2]
_LS2 = [0, 1, 2]
_BATCH = 65536


def _cg_su2(j1, m1, j2, m2, j3, m3):
    if m3 != m1 + m2:
        return 0.0
    f = math.factorial
    vmin = int(max(-j1 + j2 + m3, -j1 + m1, 0))
    vmax = int(min(j2 + j3 + m1, j3 - j1 + j2, j3 + m3))
    C = math.sqrt((2 * j3 + 1) * Fraction(
        f(j3 + j1 - j2) * f(j3 - j1 + j2) * f(j1 + j2 - j3) * f(j3 + m3) * f(j3 - m3),
        f(j1 + j2 + j3 + 1) * f(j1 - m1) * f(j1 + m1) * f(j2 - m2) * f(j2 + m2)))
    S = 0
    for v in range(vmin, vmax + 1):
        S += (-1) ** (v + j2 + m2) * Fraction(
            f(j2 + j3 + m1 - v) * f(j1 - m1 + v),
            f(v) * f(j3 - j1 + j2 - v) * f(j3 + m3 - v) * f(v + j1 - j2 - m3))
    return C * float(S)


def _cg_change_basis(l):
    q = np.zeros((2 * l + 1, 2 * l + 1), dtype=np.complex128)
    for m in range(-l, 0):
        q[l + m, l + abs(m)] = 1.0 / math.sqrt(2)
        q[l + m, l - abs(m)] = -1j / math.sqrt(2)
    q[l, l] = 1.0
    for m in range(1, l + 1):
        q[l + m, l + abs(m)] = (-1) ** m / math.sqrt(2)
        q[l + m, l - abs(m)] = 1j * (-1) ** m / math.sqrt(2)
    return (-1j) ** l * q


def _cg_real_w3j(l1, l2, l3):
    C = np.zeros((2 * l1 + 1, 2 * l2 + 1, 2 * l3 + 1), dtype=np.complex128)
    for m1 in range(-l1, l1 + 1):
        for m2 in range(-l2, l2 + 1):
            m3 = m1 + m2
            if abs(m3) <= l3:
                C[l1 + m1, l2 + m2, l3 + m3] = _cg_su2(l1, m1, l2, m2, l3, m3)
    Q1, Q2, Q3 = _cg_change_basis(l1), _cg_change_basis(l2), _cg_change_basis(l3)
    C = np.einsum('ij,kl,mn,ikn->jlm', Q1, Q2, np.conj(Q3.T), C)
    C = np.real(C)
    return C / np.linalg.norm(C)


def _cg_blocks(ls1, ls2):
    lmax2 = max(ls2)
    cb_layout = {}
    off1 = 0
    for l1 in ls1:
        off2 = 0
        for l2 in ls2:
            for l3 in range(abs(l1 - l2), l1 + l2 + 1):
                cb_layout.setdefault(l3, []).append((l1, l2, off1, off2))
            off2 += 2 * l2 + 1
        off1 += 2 * l1 + 1
    blocks = []
    row_offset = 0
    for l3 in sorted(cb_layout.keys()):
        mults = sorted(cb_layout[l3], key=lambda x: x[0] * lmax2 + x[1])
        for (l1, l2, o1, o2) in mults:
            cb = _cg_real_w3j(l1, l2, l3)
            rows = {}
            for m3 in range(2 * l3 + 1):
                terms = []
                for m2 in range(2 * l2 + 1):
                    for m1 in range(2 * l1 + 1):
                        c = cb[m1, m2, m3]
                        if abs(c) < 1e-12:
                            continue
                        terms.append((m1 + o1, m2 + o2,
                                      float(c) * math.sqrt(2 * l3 + 1)))
                if terms:
                    rows[m3 + row_offset] = terms
            blocks.append((l1, l2, rows))
            row_offset += 2 * l3 + 1
    return row_offset, blocks


_HEIGHT, _BLOCKS = _cg_blocks(_LS1, _LS2)
_DIM1 = sum(2 * l + 1 for l in _LS1)
_DIM2 = sum(2 * l + 1 for l in _LS2)

_NC, _NS, _L = 2, 16, 16
_NW = _NC * _NS
_RBLK = 256
_ROWS_PER_W = _BATCH // _NW
_NBLK = _ROWS_PER_W // _RBLK
_CHUNKS = [0, 16, 32, 48, 64, 65]


def _pair_blocks(blocks):
    merged = {}
    for (l1, l2, orows) in blocks:
        dst = merged.setdefault((l1, l2), {})
        for o, terms in orows.items():
            dst.setdefault(o, []).extend(terms)
    out = []
    for (l1, l2), orows in merged.items():
        pairs = []
        for terms in orows.values():
            for (i1, i2, _val) in terms:
                if (i1, i2) not in pairs:
                    pairs.append((i1, i2))
        out.append((pairs, orows))
    return out


_PAIR_BLOCKS = _pair_blocks(_BLOCKS)


def _tp_body(in1_hbm, in2_hbm, out_hbm, a2_v, b2_v, a_v, b_v, oc_v, op_v):
    wid = lax.axis_index("s") * _NC + lax.axis_index("c")
    base = wid * _ROWS_PER_W

    def block(bi, carry):
        rbase = base + bi * _RBLK
        pltpu.sync_copy(in1_hbm.at[pl.ds(rbase, _RBLK)], a2_v)
        pltpu.sync_copy(in2_hbm.at[pl.ds(rbase, _RBLK)], b2_v)

        @plsc.parallel_loop(0, _RBLK, 1)
        def inpack(r):
            k = lax.iota(jnp.int32, _L)
            m = k < _DIM1
            plsc.store_scatter(a_v, [r * _DIM1 + k], a2_v[r, pl.ds(0, _L)],
                               mask=m)
            plsc.store_scatter(b_v, [r * _DIM2 + k], b2_v[r, pl.ds(0, _L)],
                               mask=m)

        @plsc.parallel_loop(0, _RBLK // _L, 1)
        def group(g):
            rows = g * _L + lax.iota(jnp.int32, _L)
            rows1 = rows * _DIM1
            rows2 = rows * _DIM2
            rowso = rows * _HEIGHT
            a = [plsc.load_gather(a_v, [rows1 + i]) for i in range(_DIM1)]
            b = [plsc.load_gather(b_v, [rows2 + j]) for j in range(_DIM2)]
            for (pairs, orows) in _PAIR_BLOCKS:
                prod = {p: a[p[0]] * b[p[1]] for p in pairs}
                for o, terms in orows.items():
                    acc = None
                    for (i1, i2, val) in terms:
                        t = prod[(i1, i2)] * jnp.float32(val)
                        acc = t if acc is None else acc + t
                    plsc.store_scatter(oc_v, [rowso + o], acc)

        @plsc.parallel_loop(0, _RBLK, 1)
        def repack(r):
            for c in _CHUNKS:
                op_v[r, pl.ds(c, _L)] = oc_v[pl.ds(r * _HEIGHT + c, _L)]

        pltpu.sync_copy(op_v, out_hbm.at[pl.ds(rbase, _RBLK)])
        return carry

    lax.fori_loop(0, _NBLK, block, 0)


@jax.jit
def kernel(in1, in2):
    mesh = plsc.VectorSubcoreMesh(core_axis_name="c", subcore_axis_name="s")
    f = functools.partial(
        pl.kernel,
        mesh=mesh,
        compiler_params=pltpu.CompilerParams(
            needs_layout_passes=False,
            disable_bounds_checks=True,
            skip_device_barrier=True,
        ),
        out_type=jax.ShapeDtypeStruct((_BATCH, _HEIGHT), jnp.float32),
        scratch_types=[
            pltpu.VMEM((_RBLK, _DIM1), jnp.float32),
            pltpu.VMEM((_RBLK, _DIM2), jnp.float32),
            pltpu.VMEM((_RBLK * _DIM1,), jnp.float32),
            pltpu.VMEM((_RBLK * _DIM2,), jnp.float32),
            pltpu.VMEM((_RBLK * _HEIGHT,), jnp.float32),
            pltpu.VMEM((_RBLK, _HEIGHT), jnp.float32),
        ],
    )(_tp_body)
    return f(in1, in2)
